# Initial kernel scaffold; baseline (speedup 1.0000x reference)
import jax
import jax.numpy as jnp
from jax.experimental import pallas as pl
from jax.experimental.pallas import tpu as pltpu

_K_SPARSITY = 64
_R_BLK = 256
_C_BLK = 512


def _topk_mask_kernel(x_ref, w_ref, out_ref, acc_ref):
    j = pl.program_id(1)
    nc = pl.num_programs(1)
    s_blk = jax.lax.dot_general(
        x_ref[...], w_ref[...],
        dimension_numbers=(((1,), (1,)), ((), ())),
        preferred_element_type=jnp.float32,
    )
    off = pl.multiple_of(j * _C_BLK, _C_BLK)
    acc_ref[:, pl.ds(off, _C_BLK)] = s_blk

    @pl.when(j == nc - 1)
    def _epilogue():
        s = acc_ref[...]
        bits = jax.lax.bitcast_convert_type(s, jnp.int32)
        key = jnp.where(bits < 0, bits ^ jnp.int32(0x7FFFFFFF), bits)
        t = jnp.full((s.shape[0], 1), jnp.iinfo(jnp.int32).min, jnp.int32)
        for b in range(30, -1, -1):
            cand = t + jnp.int32(1 << b)
            cnt = jnp.sum((key >= cand).astype(jnp.int32), axis=1,
                          keepdims=True)
            t = jnp.where(cnt >= _K_SPARSITY, cand, t)
        mask = (key >= t) & (s > 0)
        out_ref[...] = jnp.where(mask, s, 0.0)


def kernel(x, W):
    B, K = x.shape
    N, K2 = W.shape
    assert K == K2 and B % _R_BLK == 0 and N % _C_BLK == 0
    grid = (B // _R_BLK, N // _C_BLK)
    return pl.pallas_call(
        _topk_mask_kernel,
        grid=grid,
        in_specs=[
            pl.BlockSpec((_R_BLK, K), lambda i, j: (i, 0)),
            pl.BlockSpec((_C_BLK, K), lambda i, j: (j, 0)),
        ],
        out_specs=pl.BlockSpec((_R_BLK, N), lambda i, j: (i, 0)),
        out_shape=jax.ShapeDtypeStruct((B, N), jnp.float32),
        scratch_shapes=[pltpu.VMEM((_R_BLK, N), jnp.float32)],
        compiler_params=pltpu.CompilerParams(
            dimension_semantics=("arbitrary", "arbitrary"),
        ),
    )(x, W)


# speedup vs baseline: 12.6967x; 12.6967x over previous
import jax
import jax.numpy as jnp
from jax.experimental import pallas as pl
from jax.experimental.pallas import tpu as pltpu

_K_SPARSITY = 64
_R_BLK = 256
_C_BLK = 512


def _topk_mask_kernel(x_ref, w_ref, out_ref, acc_ref):
    j = pl.program_id(1)
    nc = pl.num_programs(1)
    s_blk = jax.lax.dot_general(
        x_ref[...], w_ref[...],
        dimension_numbers=(((1,), (1,)), ((), ())),
        preferred_element_type=jnp.float32,
    )
    off = pl.multiple_of(j * _C_BLK, _C_BLK)
    acc_ref[:, pl.ds(off, _C_BLK)] = s_blk

    @pl.when(j == nc - 1)
    def _epilogue():
        s = acc_ref[...]
        bits = jax.lax.bitcast_convert_type(s, jnp.int32)
        key = jnp.where(bits < 0, bits ^ jnp.int32(0x7FFFFFFF), bits)
        t = jnp.full((s.shape[0], 1), jnp.iinfo(jnp.int32).min, jnp.int32)
        for b in range(31, -1, -1):
            add = jnp.int32(-(2**31)) if b == 31 else jnp.int32(1 << b)
            cand = t + add
            cnt = jnp.sum((key >= cand).astype(jnp.int32), axis=1,
                          keepdims=True)
            t = jnp.where(cnt >= _K_SPARSITY, cand, t)
        mask = (key >= t) & (s > 0)
        out_ref[...] = jnp.where(mask, s, 0.0)


def kernel(x, W):
    B, K = x.shape
    N, K2 = W.shape
    assert K == K2 and B % _R_BLK == 0 and N % _C_BLK == 0
    grid = (B // _R_BLK, N // _C_BLK)
    return pl.pallas_call(
        _topk_mask_kernel,
        grid=grid,
        in_specs=[
            pl.BlockSpec((_R_BLK, K), lambda i, j: (i, 0)),
            pl.BlockSpec((_C_BLK, K), lambda i, j: (j, 0)),
        ],
        out_specs=pl.BlockSpec((_R_BLK, N), lambda i, j: (i, 0)),
        out_shape=jax.ShapeDtypeStruct((B, N), jnp.float32),
        scratch_shapes=[pltpu.VMEM((_R_BLK, N), jnp.float32)],
        compiler_params=pltpu.CompilerParams(
            dimension_semantics=("arbitrary", "arbitrary"),
        ),
    )(x, W)
